# initial kernel scaffold (unmeasured)
import jax
import jax.numpy as jnp
from jax import lax
from jax.experimental import pallas as pl
from jax.experimental.pallas import tpu as pltpu

N_DEV = 4


def kernel(x, router_W, route_idx, expert_W, shared_W):
    n_tok, d = x.shape
    e_loc, _, h = expert_W.shape
    n_exp = router_W.shape[1]

    def body(x_ref, rw_ref, idx_ref, ew_ref, sw_ref, out_ref,
             comm_ref, send_sems, recv_sems):
        my = lax.axis_index("i")
        left = lax.rem(my + N_DEV - 1, N_DEV)
        right = lax.rem(my + 1, N_DEV)

        barrier_sem = pltpu.get_barrier_semaphore()
        for nbr in (left, right):
            pl.semaphore_signal(barrier_sem, inc=1, device_id=(nbr,),
                                device_id_type=pl.DeviceIdType.MESH)
        pl.semaphore_wait(barrier_sem, 2)

        rdma = pltpu.make_async_remote_copy(
            src_ref=ew_ref, dst_ref=comm_ref.at[0],
            send_sem=send_sems.at[0], recv_sem=recv_sems.at[0],
            device_id=(right,), device_id_type=pl.DeviceIdType.MESH)
        rdma.start()

        xv = x_ref[...]
        idx = idx_ref[...]

        scores = jnp.dot(xv, rw_ref[...], preferred_element_type=jnp.float32)
        s_max = jnp.max(scores, axis=-1, keepdims=True)
        p = jnp.exp(scores - s_max)
        probs = p / jnp.sum(p, axis=-1, keepdims=True)
        e_iota = lax.broadcasted_iota(jnp.int32, (n_tok, n_exp), 1)
        p_chosen = jnp.sum(jnp.where(idx == e_iota, probs, 0.0),
                           axis=1, keepdims=True)

        out_ref[...] = jnp.dot(xv, sw_ref[...],
                               preferred_element_type=jnp.float32)

        def accum(block, origin):
            W = block.reshape(e_loc * d, h)
            parts = []
            for j in range(e_loc):
                ge = origin * e_loc + j
                sel = jnp.where(idx == ge, p_chosen, 0.0)
                parts.append(xv * sel)
            xb = jnp.concatenate(parts, axis=1)
            out_ref[...] += jnp.dot(xb, W, preferred_element_type=jnp.float32)

        accum(ew_ref[...], my)

        for hop in range(1, N_DEV - 1):
            rdma.wait()
            rdma = pltpu.make_async_remote_copy(
                src_ref=comm_ref.at[hop - 1], dst_ref=comm_ref.at[hop],
                send_sem=send_sems.at[hop], recv_sem=recv_sems.at[hop],
                device_id=(right,), device_id_type=pl.DeviceIdType.MESH)
            rdma.start()
            accum(comm_ref[hop - 1], lax.rem(my + N_DEV - hop, N_DEV))
        rdma.wait()
        accum(comm_ref[N_DEV - 2], lax.rem(my + 1, N_DEV))

    out_shape = jax.ShapeDtypeStruct((n_tok, h), jnp.float32)
    return pl.pallas_call(
        body,
        out_shape=out_shape,
        in_specs=[pl.BlockSpec(memory_space=pltpu.VMEM)] * 5,
        out_specs=pl.BlockSpec(memory_space=pltpu.VMEM),
        scratch_shapes=[
            pltpu.VMEM((N_DEV - 1, e_loc, d, h), jnp.float32),
            pltpu.SemaphoreType.DMA((N_DEV - 1,)),
            pltpu.SemaphoreType.DMA((N_DEV - 1,)),
        ],
        compiler_params=pltpu.CompilerParams(collective_id=0),
    )(x, router_W, route_idx, expert_W, shared_W)


# baseline (device time: 306859 ns/iter reference)
import jax
import jax.numpy as jnp
from jax import lax
from jax.experimental import pallas as pl
from jax.experimental.pallas import tpu as pltpu

N_DEV = 4


def kernel(x, router_W, route_idx, expert_W, shared_W):
    n_tok, d = x.shape
    e_loc, _, h = expert_W.shape
    n_exp = router_W.shape[1]

    def body(x_ref, rw_ref, idx_ref, ew_ref, sw_ref, out_ref,
             comm_ref, send_sems, recv_sems):
        my = lax.axis_index("i")
        left = lax.rem(my + N_DEV - 1, N_DEV)
        right = lax.rem(my + 1, N_DEV)

        barrier_sem = pltpu.get_barrier_semaphore()
        for nbr in (left, right):
            pl.semaphore_signal(barrier_sem, inc=1, device_id=(nbr,),
                                device_id_type=pl.DeviceIdType.MESH)
        pl.semaphore_wait(barrier_sem, 2)

        rdma = pltpu.make_async_remote_copy(
            src_ref=ew_ref, dst_ref=comm_ref.at[0],
            send_sem=send_sems.at[0], recv_sem=recv_sems.at[0],
            device_id=(right,), device_id_type=pl.DeviceIdType.MESH)
        rdma.start()

        xv = x_ref[...]
        idx = idx_ref[...]

        scores = jnp.dot(xv, rw_ref[...], preferred_element_type=jnp.float32)
        s_max = jnp.max(scores, axis=-1, keepdims=True)
        p = jnp.exp(scores - s_max)
        probs = p / jnp.sum(p, axis=-1, keepdims=True)
        e_iota = lax.broadcasted_iota(jnp.int32, (n_tok, n_exp), 1)
        p_chosen = jnp.sum(jnp.where(idx == e_iota, probs, 0.0),
                           axis=1, keepdims=True)

        out_ref[...] = jnp.dot(xv, sw_ref[...],
                               preferred_element_type=jnp.float32)

        def accum(block_ref, origin):
            for j in range(e_loc):
                ge = origin * e_loc + j
                sel = jnp.where(idx == ge, p_chosen, 0.0)
                out_ref[...] += jnp.dot(xv * sel, block_ref[j],
                                        preferred_element_type=jnp.float32)

        accum(ew_ref, my)

        for hop in range(1, N_DEV - 1):
            rdma.wait()
            rdma = pltpu.make_async_remote_copy(
                src_ref=comm_ref.at[hop - 1], dst_ref=comm_ref.at[hop],
                send_sem=send_sems.at[hop], recv_sem=recv_sems.at[hop],
                device_id=(right,), device_id_type=pl.DeviceIdType.MESH)
            rdma.start()
            accum(comm_ref.at[hop - 1], lax.rem(my + N_DEV - hop, N_DEV))
        rdma.wait()
        accum(comm_ref.at[N_DEV - 2], lax.rem(my + 1, N_DEV))

    out_shape = jax.ShapeDtypeStruct((n_tok, h), jnp.float32)
    return pl.pallas_call(
        body,
        out_shape=out_shape,
        in_specs=[pl.BlockSpec(memory_space=pltpu.VMEM)] * 5,
        out_specs=pl.BlockSpec(memory_space=pltpu.VMEM),
        scratch_shapes=[
            pltpu.VMEM((N_DEV - 1, e_loc, d, h), jnp.float32),
            pltpu.SemaphoreType.DMA((N_DEV - 1,)),
            pltpu.SemaphoreType.DMA((N_DEV - 1,)),
        ],
        compiler_params=pltpu.CompilerParams(
            collective_id=0, vmem_limit_bytes=100 * 1024 * 1024),
    )(x, router_W, route_idx, expert_W, shared_W)


# device time: 104622 ns/iter; 2.9330x vs baseline; 2.9330x over previous
import jax
import jax.numpy as jnp
from jax import lax
from jax.experimental import pallas as pl
from jax.experimental.pallas import tpu as pltpu

N_DEV = 4
E_HALF = 2


def kernel(x, router_W, route_idx, expert_W, shared_W):
    n_tok, d = x.shape
    e_loc, _, h = expert_W.shape
    n_exp = router_W.shape[1]

    def body(x_ref, rw_ref, idx_ref, ew_ref, sw_ref, out_ref,
             cw_buf, ccw_buf, cw_send, cw_recv, ccw_send, ccw_recv):
        my = lax.axis_index("i")
        left = lax.rem(my + N_DEV - 1, N_DEV)
        right = lax.rem(my + 1, N_DEV)

        cw_buf[0] = ew_ref[0:E_HALF].astype(jnp.bfloat16)
        ccw_buf[0] = ew_ref[E_HALF:2 * E_HALF].astype(jnp.bfloat16)

        barrier_sem = pltpu.get_barrier_semaphore()
        for nbr in (left, right):
            pl.semaphore_signal(barrier_sem, inc=1, device_id=(nbr,),
                                device_id_type=pl.DeviceIdType.MESH)
        pl.semaphore_wait(barrier_sem, 2)

        def start_hop(hop):
            cw = pltpu.make_async_remote_copy(
                src_ref=cw_buf.at[hop], dst_ref=cw_buf.at[hop + 1],
                send_sem=cw_send.at[hop], recv_sem=cw_recv.at[hop],
                device_id=(right,), device_id_type=pl.DeviceIdType.MESH)
            ccw = pltpu.make_async_remote_copy(
                src_ref=ccw_buf.at[hop], dst_ref=ccw_buf.at[hop + 1],
                send_sem=ccw_send.at[hop], recv_sem=ccw_recv.at[hop],
                device_id=(left,), device_id_type=pl.DeviceIdType.MESH)
            cw.start()
            ccw.start()
            return cw, ccw

        cw, ccw = start_hop(0)

        xv = x_ref[...]
        xb = xv.astype(jnp.bfloat16)
        idx = idx_ref[...]

        scores = jnp.dot(xv, rw_ref[...], preferred_element_type=jnp.float32)
        s_max = jnp.max(scores, axis=-1, keepdims=True)
        p = jnp.exp(scores - s_max)
        probs = p / jnp.sum(p, axis=-1, keepdims=True)
        e_iota = lax.broadcasted_iota(jnp.int32, (n_tok, n_exp), 1)
        p_chosen = jnp.sum(jnp.where(idx == e_iota, probs, 0.0),
                           axis=1, keepdims=True)

        out_ref[...] = jnp.dot(xb, sw_ref[...].astype(jnp.bfloat16),
                               preferred_element_type=jnp.float32)

        def accum(buf, slot, origin, j_base):
            for j in range(E_HALF):
                ge = origin * e_loc + j_base + j
                sel = jnp.where(idx == ge, p_chosen, 0.0)
                xm = xb * sel.astype(jnp.bfloat16)
                out_ref[...] += jnp.dot(xm, buf[slot, j],
                                        preferred_element_type=jnp.float32)

        accum(cw_buf, 0, my, 0)
        accum(ccw_buf, 0, my, E_HALF)

        for hop in range(1, N_DEV - 1):
            cw.wait()
            ccw.wait()
            cw, ccw = start_hop(hop)
            accum(cw_buf, hop, lax.rem(my + N_DEV - hop, N_DEV), 0)
            accum(ccw_buf, hop, lax.rem(my + hop, N_DEV), E_HALF)
        cw.wait()
        ccw.wait()
        accum(cw_buf, N_DEV - 1, lax.rem(my + 1, N_DEV), 0)
        accum(ccw_buf, N_DEV - 1, lax.rem(my + N_DEV - 1, N_DEV), E_HALF)

    out_shape = jax.ShapeDtypeStruct((n_tok, h), jnp.float32)
    return pl.pallas_call(
        body,
        out_shape=out_shape,
        in_specs=[pl.BlockSpec(memory_space=pltpu.VMEM)] * 5,
        out_specs=pl.BlockSpec(memory_space=pltpu.VMEM),
        scratch_shapes=[
            pltpu.VMEM((N_DEV, E_HALF, d, h), jnp.bfloat16),
            pltpu.VMEM((N_DEV, E_HALF, d, h), jnp.bfloat16),
            pltpu.SemaphoreType.DMA((N_DEV - 1,)),
            pltpu.SemaphoreType.DMA((N_DEV - 1,)),
            pltpu.SemaphoreType.DMA((N_DEV - 1,)),
            pltpu.SemaphoreType.DMA((N_DEV - 1,)),
        ],
        compiler_params=pltpu.CompilerParams(
            collective_id=0, vmem_limit_bytes=100 * 1024 * 1024),
    )(x, router_W, route_idx, expert_W, shared_W)


# device time: 97573 ns/iter; 3.1449x vs baseline; 1.0722x over previous
import jax
import jax.numpy as jnp
from jax import lax
from jax.experimental import pallas as pl
from jax.experimental.pallas import tpu as pltpu

N_DEV = 4
E_HALF = 2
H_SPLIT = 2
N_CHUNK = E_HALF * H_SPLIT


def kernel(x, router_W, route_idx, expert_W, shared_W):
    n_tok, d = x.shape
    e_loc, _, h = expert_W.shape
    n_exp = router_W.shape[1]
    h2 = h // H_SPLIT

    def body(x_ref, rw_ref, idx_ref, ew_ref, sw_ref, out_ref,
             cw_buf, ccw_buf, cw_send, cw_recv, ccw_send, ccw_recv):
        my = lax.axis_index("i")
        left = lax.rem(my + N_DEV - 1, N_DEV)
        right = lax.rem(my + 1, N_DEV)

        for j in range(E_HALF):
            for k in range(H_SPLIT):
                q = H_SPLIT * j + k
                cw_buf[0, q] = (
                    ew_ref[j, :, k * h2:(k + 1) * h2].astype(jnp.bfloat16))
                ccw_buf[0, q] = (
                    ew_ref[E_HALF + j, :, k * h2:(k + 1) * h2]
                    .astype(jnp.bfloat16))

        barrier_sem = pltpu.get_barrier_semaphore()
        for nbr in (left, right):
            pl.semaphore_signal(barrier_sem, inc=1, device_id=(nbr,),
                                device_id_type=pl.DeviceIdType.MESH)
        pl.semaphore_wait(barrier_sem, 2)

        def mk(buf, send_sems, recv_sems, t, q, dst):
            return pltpu.make_async_remote_copy(
                src_ref=buf.at[t, q], dst_ref=buf.at[t + 1, q],
                send_sem=send_sems.at[t, q], recv_sem=recv_sems.at[t, q],
                device_id=(dst,), device_id_type=pl.DeviceIdType.MESH)

        cw_d, ccw_d = {}, {}
        for q in range(N_CHUNK):
            cw_d[0, q] = mk(cw_buf, cw_send, cw_recv, 0, q, right)
            ccw_d[0, q] = mk(ccw_buf, ccw_send, ccw_recv, 0, q, left)
            cw_d[0, q].start()
            ccw_d[0, q].start()

        xv = x_ref[...]
        xb = xv.astype(jnp.bfloat16)
        idx = idx_ref[...]

        scores = jnp.dot(xv, rw_ref[...], preferred_element_type=jnp.float32)
        s_max = jnp.max(scores, axis=-1, keepdims=True)
        p = jnp.exp(scores - s_max)
        probs = p / jnp.sum(p, axis=-1, keepdims=True)
        e_iota = lax.broadcasted_iota(jnp.int32, (n_tok, n_exp), 1)
        p_chosen = jnp.sum(jnp.where(idx == e_iota, probs, 0.0),
                           axis=1, keepdims=True)

        out_ref[...] = jnp.dot(xb, sw_ref[...].astype(jnp.bfloat16),
                               preferred_element_type=jnp.float32)

        def accum_expert(buf, slot, origin, j_base, j):
            ge = origin * e_loc + j_base + j
            sel = jnp.where(idx == ge, p_chosen, 0.0)
            xm = xb * sel.astype(jnp.bfloat16)
            for k in range(H_SPLIT):
                out_ref[:, k * h2:(k + 1) * h2] += jnp.dot(
                    xm, buf[slot, H_SPLIT * j + k],
                    preferred_element_type=jnp.float32)

        def accum(buf, slot, origin, j_base):
            for j in range(E_HALF):
                accum_expert(buf, slot, origin, j_base, j)

        accum(cw_buf, 0, my, 0)
        accum(ccw_buf, 0, my, E_HALF)

        for t in range(1, N_DEV - 1):
            for q in range(N_CHUNK):
                cw_d[t - 1, q].wait_recv()
                cw_d[t, q] = mk(cw_buf, cw_send, cw_recv, t, q, right)
                cw_d[t, q].start()
                ccw_d[t - 1, q].wait_recv()
                ccw_d[t, q] = mk(ccw_buf, ccw_send, ccw_recv, t, q, left)
                ccw_d[t, q].start()
            accum(cw_buf, t, lax.rem(my + N_DEV - t, N_DEV), 0)
            accum(ccw_buf, t, lax.rem(my + t, N_DEV), E_HALF)

        for j in range(E_HALF):
            for k in range(H_SPLIT):
                cw_d[N_DEV - 2, H_SPLIT * j + k].wait_recv()
            accum_expert(cw_buf, N_DEV - 1, lax.rem(my + 1, N_DEV), 0, j)
            for k in range(H_SPLIT):
                ccw_d[N_DEV - 2, H_SPLIT * j + k].wait_recv()
            accum_expert(ccw_buf, N_DEV - 1, lax.rem(my + N_DEV - 1, N_DEV),
                         E_HALF, j)

        for t in range(N_DEV - 1):
            for q in range(N_CHUNK):
                cw_d[t, q].wait_send()
                ccw_d[t, q].wait_send()

    out_shape = jax.ShapeDtypeStruct((n_tok, h), jnp.float32)
    return pl.pallas_call(
        body,
        out_shape=out_shape,
        in_specs=[pl.BlockSpec(memory_space=pltpu.VMEM)] * 5,
        out_specs=pl.BlockSpec(memory_space=pltpu.VMEM),
        scratch_shapes=[
            pltpu.VMEM((N_DEV, N_CHUNK, d, h2), jnp.bfloat16),
            pltpu.VMEM((N_DEV, N_CHUNK, d, h2), jnp.bfloat16),
            pltpu.SemaphoreType.DMA((N_DEV - 1, N_CHUNK)),
            pltpu.SemaphoreType.DMA((N_DEV - 1, N_CHUNK)),
            pltpu.SemaphoreType.DMA((N_DEV - 1, N_CHUNK)),
            pltpu.SemaphoreType.DMA((N_DEV - 1, N_CHUNK)),
        ],
        compiler_params=pltpu.CompilerParams(
            collective_id=0, vmem_limit_bytes=100 * 1024 * 1024),
    )(x, router_W, route_idx, expert_W, shared_W)
